# Initial kernel scaffold; baseline (speedup 1.0000x reference)
import jax
import jax.numpy as jnp
from jax.experimental import pallas as pl


def kernel(hidden_states, W0, b0, Wdiff, bdiff, orig_v):
    raise NotImplementedError("write your pallas kernel here")


# speedup vs baseline: 1.9564x; 1.9564x over previous
import functools

import jax
import jax.numpy as jnp
from jax.experimental import pallas as pl
from jax.experimental.pallas import tpu as pltpu


def _gate_kernel(x_ref, v_ref, vt_ref, rw_ref, *, n_exp, gate_k):
    x = x_ref[...]
    x16 = x.astype(jnp.bfloat16)
    coef = jnp.dot(x16, v_ref[...], preferred_element_type=jnp.float32)
    coef16 = coef.astype(jnp.bfloat16)
    res_cols = []
    for e in range(n_exp):
        pe = jnp.dot(coef16[:, e * gate_k:(e + 1) * gate_k],
                     vt_ref[e * gate_k:(e + 1) * gate_k, :],
                     preferred_element_type=jnp.float32)
        de = x - pe
        res_cols.append(jnp.sum(de * de, axis=1, keepdims=True))
    res = jnp.sqrt(jnp.concatenate(res_cols, axis=1))
    m = jnp.max(-res, axis=1, keepdims=True)
    e = jnp.exp(-res - m)
    rw_ref[...] = e / jnp.sum(e, axis=1, keepdims=True)


def _combine_kernel(any_ref, x_ref, rw_ref, w0_ref, b0_ref, wd_ref, bd_ref,
                    out_ref, *, n_exp, top_k):
    rw = rw_ref[...]
    ids = jax.lax.broadcasted_iota(jnp.int32, rw.shape, 1)
    thresh_f = (rw > (1.0 / n_exp)).astype(rw.dtype)
    mx1 = jnp.max(rw, axis=1, keepdims=True)
    i1 = jnp.min(jnp.where(rw == mx1, ids, n_exp), axis=1, keepdims=True)
    fb_f = (ids == i1).astype(rw.dtype)
    any_flag = any_ref[0] > 0
    base_f = jnp.where(any_flag, thresh_f, fb_f)
    tk_f = fb_f
    cur = jnp.where(ids == i1, -jnp.inf, rw)
    for _ in range(top_k - 1):
        mxk = jnp.max(cur, axis=1, keepdims=True)
        ik = jnp.min(jnp.where(cur == mxk, ids, n_exp), axis=1, keepdims=True)
        tk_f = tk_f + (ids == ik).astype(rw.dtype)
        cur = jnp.where(ids == ik, -jnp.inf, cur)
    filt = rw * base_f * tk_f
    sw = jnp.sum(filt, axis=1, keepdims=True)
    sw = jnp.where(sw == 0.0, 1.0, sw)
    nw = filt / sw

    x = x_ref[...]
    acc = jnp.dot(x, w0_ref[...], preferred_element_type=jnp.float32)
    acc = acc + b0_ref[...]
    acc = acc + jnp.dot(nw, bd_ref[...], preferred_element_type=jnp.float32)
    for e in range(n_exp):
        pe = jnp.dot(x, wd_ref[e], preferred_element_type=jnp.float32)
        acc = acc + nw[:, e:e + 1] * pe
    out_ref[...] = acc


def kernel(hidden_states, W0, b0, Wdiff, bdiff, orig_v):
    B, S, D_IN = hidden_states.shape
    E, D_OUT, _ = Wdiff.shape
    GK = orig_v.shape[2]
    TOP_K = 2
    T = B * S

    x = hidden_states.reshape(T, D_IN)
    v_flat = jnp.transpose(orig_v, (1, 0, 2)).reshape(D_IN, E * GK)
    v16 = v_flat.astype(jnp.bfloat16)
    vt16 = v_flat.T.astype(jnp.bfloat16)

    TBG = 1024
    rw = pl.pallas_call(
        functools.partial(_gate_kernel, n_exp=E, gate_k=GK),
        grid=(T // TBG,),
        in_specs=[
            pl.BlockSpec((TBG, D_IN), lambda i: (i, 0)),
            pl.BlockSpec((D_IN, E * GK), lambda i: (0, 0)),
            pl.BlockSpec((E * GK, D_IN), lambda i: (0, 0)),
        ],
        out_specs=pl.BlockSpec((TBG, E), lambda i: (i, 0)),
        out_shape=jax.ShapeDtypeStruct((T, E), jnp.float32),
    )(x, v16, vt16)

    any_flag = jnp.any(rw > (1.0 / E)).astype(jnp.int32).reshape(1)

    x16 = x.astype(jnp.bfloat16)
    w0t = W0.T.astype(jnp.bfloat16)
    wdt = jnp.transpose(Wdiff, (0, 2, 1)).astype(jnp.bfloat16)
    b0r = b0.reshape(1, D_OUT)

    TB = 512
    out = pl.pallas_call(
        functools.partial(_combine_kernel, n_exp=E, top_k=TOP_K),
        grid=(T // TB,),
        in_specs=[
            pl.BlockSpec(memory_space=pltpu.SMEM),
            pl.BlockSpec((TB, D_IN), lambda i: (i, 0)),
            pl.BlockSpec((TB, E), lambda i: (i, 0)),
            pl.BlockSpec((D_IN, D_OUT), lambda i: (0, 0)),
            pl.BlockSpec((1, D_OUT), lambda i: (0, 0)),
            pl.BlockSpec((E, D_IN, D_OUT), lambda i: (0, 0, 0)),
            pl.BlockSpec((E, D_OUT), lambda i: (0, 0)),
        ],
        out_specs=pl.BlockSpec((TB, D_OUT), lambda i: (i, 0)),
        out_shape=jax.ShapeDtypeStruct((T, D_OUT), jnp.float32),
    )(any_flag, x16, rw, w0t, b0r, wdt, bdiff)

    return out.reshape(B, S, D_OUT)


# speedup vs baseline: 2.2837x; 1.1673x over previous
import functools

import jax
import jax.numpy as jnp
from jax.experimental import pallas as pl
from jax.experimental.pallas import tpu as pltpu


def _fused_kernel(x_ref, v_ref, vt_ref, w0_ref, b0_ref, wd_ref, bd_ref,
                  out_ref, x16_s, rw_s, w016_s, wd16_s, any_s,
                  *, n_exp, gate_k, top_k, nb, tb):
    i = pl.program_id(0)

    @pl.when(i == 0)
    def _init():
        any_s[0] = 0

    @pl.when(i < nb)
    def _gate_phase():
        x = x_ref[...]
        x16 = x.astype(jnp.bfloat16)
        coef = jnp.dot(x16, v_ref[...], preferred_element_type=jnp.float32)
        coef16 = coef.astype(jnp.bfloat16)
        res_cols = []
        for e in range(n_exp):
            pe = jnp.dot(coef16[:, e * gate_k:(e + 1) * gate_k],
                         vt_ref[e * gate_k:(e + 1) * gate_k, :],
                         preferred_element_type=jnp.float32)
            de = x - pe
            res_cols.append(jnp.sum(de * de, axis=1, keepdims=True))
        res = jnp.sqrt(jnp.concatenate(res_cols, axis=1))
        m = jnp.max(-res, axis=1, keepdims=True)
        ex = jnp.exp(-res - m)
        rw = ex / jnp.sum(ex, axis=1, keepdims=True)
        x16_s[pl.ds(i * tb, tb), :] = x16
        rw_s[pl.ds(i * tb, tb), :] = rw
        blk_any = jnp.max((rw > (1.0 / n_exp)).astype(jnp.int32))
        any_s[0] = jnp.maximum(any_s[0], blk_any)

    @pl.when(i == nb)
    def _cast_weights():
        w016_s[...] = w0_ref[...].astype(jnp.bfloat16)
        wd16_s[...] = wd_ref[...].astype(jnp.bfloat16)

    @pl.when(i >= nb)
    def _combine_phase():
        j = i - nb
        rw = rw_s[pl.ds(j * tb, tb), :]
        ids = jax.lax.broadcasted_iota(jnp.int32, rw.shape, 1)
        thresh_f = (rw > (1.0 / n_exp)).astype(rw.dtype)
        mx1 = jnp.max(rw, axis=1, keepdims=True)
        i1 = jnp.min(jnp.where(rw == mx1, ids, n_exp), axis=1, keepdims=True)
        fb_f = (ids == i1).astype(rw.dtype)
        base_f = jnp.where(any_s[0] > 0, thresh_f, fb_f)
        tk_f = fb_f
        cur = jnp.where(ids == i1, -jnp.inf, rw)
        for _ in range(top_k - 1):
            mxk = jnp.max(cur, axis=1, keepdims=True)
            ik = jnp.min(jnp.where(cur == mxk, ids, n_exp), axis=1,
                         keepdims=True)
            tk_f = tk_f + (ids == ik).astype(rw.dtype)
            cur = jnp.where(ids == ik, -jnp.inf, cur)
        filt = rw * base_f * tk_f
        sw = jnp.sum(filt, axis=1, keepdims=True)
        sw = jnp.where(sw == 0.0, 1.0, sw)
        nw = filt / sw

        x16 = x16_s[pl.ds(j * tb, tb), :]
        dn = (((1,), (1,)), ((), ()))
        acc = jax.lax.dot_general(x16, w016_s[...], dn,
                                  preferred_element_type=jnp.float32)
        acc = acc + b0_ref[...]
        acc = acc + jnp.dot(nw, bd_ref[...],
                            preferred_element_type=jnp.float32)
        for e in range(n_exp):
            pe = jax.lax.dot_general(x16, wd16_s[e], dn,
                                     preferred_element_type=jnp.float32)
            acc = acc + nw[:, e:e + 1] * pe
        out_ref[...] = acc


def kernel(hidden_states, W0, b0, Wdiff, bdiff, orig_v):
    B, S, D_IN = hidden_states.shape
    E, D_OUT, _ = Wdiff.shape
    GK = orig_v.shape[2]
    TOP_K = 2
    T = B * S

    x = hidden_states.reshape(T, D_IN)
    v_flat = jnp.transpose(orig_v, (1, 0, 2)).reshape(D_IN, E * GK)
    v16 = v_flat.astype(jnp.bfloat16)
    vt16 = v_flat.T.astype(jnp.bfloat16)
    b0r = b0.reshape(1, D_OUT)

    TB = 512
    NB = T // TB
    out = pl.pallas_call(
        functools.partial(_fused_kernel, n_exp=E, gate_k=GK, top_k=TOP_K,
                          nb=NB, tb=TB),
        grid=(2 * NB,),
        in_specs=[
            pl.BlockSpec((TB, D_IN), lambda i, _nb=NB: (jnp.minimum(i, _nb - 1), 0)),
            pl.BlockSpec((D_IN, E * GK), lambda i: (0, 0)),
            pl.BlockSpec((E * GK, D_IN), lambda i: (0, 0)),
            pl.BlockSpec((D_OUT, D_IN), lambda i: (0, 0)),
            pl.BlockSpec((1, D_OUT), lambda i: (0, 0)),
            pl.BlockSpec((E, D_OUT, D_IN), lambda i: (0, 0, 0)),
            pl.BlockSpec((E, D_OUT), lambda i: (0, 0)),
        ],
        out_specs=pl.BlockSpec((TB, D_OUT), lambda i, _nb=NB: (jnp.maximum(i - _nb, 0), 0)),
        out_shape=jax.ShapeDtypeStruct((T, D_OUT), jnp.float32),
        scratch_shapes=[
            pltpu.VMEM((T, D_IN), jnp.bfloat16),
            pltpu.VMEM((T, E), jnp.float32),
            pltpu.VMEM((D_OUT, D_IN), jnp.bfloat16),
            pltpu.VMEM((E, D_OUT, D_IN), jnp.bfloat16),
            pltpu.SMEM((1,), jnp.int32),
        ],
        compiler_params=pltpu.CompilerParams(
            dimension_semantics=("arbitrary",)),
    )(x, v16, vt16, W0, b0r, Wdiff, bdiff)

    return out.reshape(B, S, D_OUT)


# speedup vs baseline: 2.3059x; 1.0097x over previous
import functools

import jax
import jax.numpy as jnp
from jax.experimental import pallas as pl
from jax.experimental.pallas import tpu as pltpu


def _moe_kernel(x_ref, v_ref, vt_ref, w0_ref, b0_ref, wd_ref, bd_ref,
                out_ref, any_ref, *, n_exp, gate_k, top_k, assume_any):
    x = x_ref[...]
    x16 = x.astype(jnp.bfloat16)
    coef = jnp.dot(x16, v_ref[...], preferred_element_type=jnp.float32)
    coef16 = coef.astype(jnp.bfloat16)
    res_cols = []
    for e in range(n_exp):
        pe = jnp.dot(coef16[:, e * gate_k:(e + 1) * gate_k],
                     vt_ref[e * gate_k:(e + 1) * gate_k, :],
                     preferred_element_type=jnp.float32)
        de = x - pe
        res_cols.append(jnp.sum(de * de, axis=1, keepdims=True))
    res = jnp.sqrt(jnp.concatenate(res_cols, axis=1))
    m = jnp.max(-res, axis=1, keepdims=True)
    ex = jnp.exp(-res - m)
    rw = ex / jnp.sum(ex, axis=1, keepdims=True)

    ids = jax.lax.broadcasted_iota(jnp.int32, rw.shape, 1)
    thresh_f = (rw > (1.0 / n_exp)).astype(rw.dtype)
    any_ref[...] = jnp.broadcast_to(jnp.max(thresh_f), any_ref.shape)
    mx1 = jnp.max(rw, axis=1, keepdims=True)
    i1 = jnp.min(jnp.where(rw == mx1, ids, n_exp), axis=1, keepdims=True)
    fb_f = (ids == i1).astype(rw.dtype)
    base_f = thresh_f if assume_any else fb_f
    tk_f = fb_f
    cur = jnp.where(ids == i1, -jnp.inf, rw)
    for _ in range(top_k - 1):
        mxk = jnp.max(cur, axis=1, keepdims=True)
        ik = jnp.min(jnp.where(cur == mxk, ids, n_exp), axis=1, keepdims=True)
        tk_f = tk_f + (ids == ik).astype(rw.dtype)
        cur = jnp.where(ids == ik, -jnp.inf, cur)
    filt = rw * base_f * tk_f
    sw = jnp.sum(filt, axis=1, keepdims=True)
    sw = jnp.where(sw == 0.0, 1.0, sw)
    nw = filt / sw

    dn = (((1,), (1,)), ((), ()))
    acc = jax.lax.dot_general(x16, w0_ref[...], dn,
                              preferred_element_type=jnp.float32)
    acc = acc + b0_ref[...]
    acc = acc + jnp.dot(nw, bd_ref[...], preferred_element_type=jnp.float32)
    for e in range(n_exp):
        pe = jax.lax.dot_general(x16, wd_ref[e], dn,
                                 preferred_element_type=jnp.float32)
        acc = acc + nw[:, e:e + 1] * pe
    out_ref[...] = acc


def kernel(hidden_states, W0, b0, Wdiff, bdiff, orig_v):
    B, S, D_IN = hidden_states.shape
    E, D_OUT, _ = Wdiff.shape
    GK = orig_v.shape[2]
    TOP_K = 2
    T = B * S
    TB = 512
    NB = T // TB

    x = hidden_states.reshape(T, D_IN)
    v_flat = jnp.transpose(orig_v, (1, 0, 2)).reshape(D_IN, E * GK)
    v16 = v_flat.astype(jnp.bfloat16)
    vt16 = v_flat.T.astype(jnp.bfloat16)
    w016 = W0.astype(jnp.bfloat16)
    wd16 = Wdiff.astype(jnp.bfloat16)
    b0r = b0.reshape(1, D_OUT)

    def run(assume_any):
        return pl.pallas_call(
            functools.partial(_moe_kernel, n_exp=E, gate_k=GK, top_k=TOP_K,
                              assume_any=assume_any),
            grid=(NB,),
            in_specs=[
                pl.BlockSpec((TB, D_IN), lambda i: (i, 0)),
                pl.BlockSpec((D_IN, E * GK), lambda i: (0, 0)),
                pl.BlockSpec((E * GK, D_IN), lambda i: (0, 0)),
                pl.BlockSpec((D_OUT, D_IN), lambda i: (0, 0)),
                pl.BlockSpec((1, D_OUT), lambda i: (0, 0)),
                pl.BlockSpec((E, D_OUT, D_IN), lambda i: (0, 0, 0)),
                pl.BlockSpec((E, D_OUT), lambda i: (0, 0)),
            ],
            out_specs=[
                pl.BlockSpec((TB, D_OUT), lambda i: (i, 0)),
                pl.BlockSpec((1, 1, 128), lambda i: (i, 0, 0)),
            ],
            out_shape=[
                jax.ShapeDtypeStruct((T, D_OUT), jnp.float32),
                jax.ShapeDtypeStruct((NB, 1, 128), jnp.float32),
            ],
            compiler_params=pltpu.CompilerParams(
                dimension_semantics=("arbitrary",)),
        )(x, v16, vt16, w016, b0r, wd16, bdiff)

    out_main, any_arr = run(True)
    any_flag = jnp.max(any_arr) > 0.0
    out = jax.lax.cond(any_flag,
                       lambda: out_main,
                       lambda: run(False)[0])
    return out.reshape(B, S, D_OUT)


# speedup vs baseline: 2.3933x; 1.0379x over previous
import functools

import jax
import jax.numpy as jnp
from jax.experimental import pallas as pl
from jax.experimental.pallas import tpu as pltpu


def _moe_kernel(x_ref, v_ref, vt_ref, w0_ref, b0_ref, wd_ref, bd_ref,
                out_ref, any_ref, *, n_exp, gate_k, top_k, assume_any):
    x = x_ref[...]
    x16 = x.astype(jnp.bfloat16)
    coef = jnp.dot(x16, v_ref[...], preferred_element_type=jnp.float32)
    coef16 = coef.astype(jnp.bfloat16)
    res_cols = []
    for e in range(n_exp):
        pe = jnp.dot(coef16[:, e * gate_k:(e + 1) * gate_k],
                     vt_ref[e * gate_k:(e + 1) * gate_k, :],
                     preferred_element_type=jnp.float32)
        de = x - pe
        res_cols.append(jnp.sum(de * de, axis=1, keepdims=True))
    res = jnp.sqrt(jnp.concatenate(res_cols, axis=1))
    m = jnp.max(-res, axis=1, keepdims=True)
    ex = jnp.exp(-res - m)
    rw = ex / jnp.sum(ex, axis=1, keepdims=True)

    ids = jax.lax.broadcasted_iota(jnp.int32, rw.shape, 1)
    thresh_f = (rw > (1.0 / n_exp)).astype(rw.dtype)
    any_ref[...] = jnp.broadcast_to(jnp.max(thresh_f), any_ref.shape)
    mx1 = jnp.max(rw, axis=1, keepdims=True)
    i1 = jnp.min(jnp.where(rw == mx1, ids, n_exp), axis=1, keepdims=True)
    fb_f = (ids == i1).astype(rw.dtype)
    base_f = thresh_f if assume_any else fb_f
    tk_f = fb_f
    cur = jnp.where(ids == i1, -jnp.inf, rw)
    for _ in range(top_k - 1):
        mxk = jnp.max(cur, axis=1, keepdims=True)
        ik = jnp.min(jnp.where(cur == mxk, ids, n_exp), axis=1, keepdims=True)
        tk_f = tk_f + (ids == ik).astype(rw.dtype)
        cur = jnp.where(ids == ik, -jnp.inf, cur)
    filt = rw * base_f * tk_f
    sw = jnp.sum(filt, axis=1, keepdims=True)
    sw = jnp.where(sw == 0.0, 1.0, sw)
    nw = filt / sw

    dn = (((1,), (1,)), ((), ()))
    acc = jax.lax.dot_general(x16, w0_ref[...], dn,
                              preferred_element_type=jnp.float32)
    acc = acc + b0_ref[...]
    acc = acc + jnp.dot(nw, bd_ref[...], preferred_element_type=jnp.float32)
    for e in range(n_exp):
        pe = jax.lax.dot_general(x16, wd_ref[e], dn,
                                 preferred_element_type=jnp.float32)
        acc = acc + nw[:, e:e + 1] * pe
    out_ref[...] = acc


def kernel(hidden_states, W0, b0, Wdiff, bdiff, orig_v):
    B, S, D_IN = hidden_states.shape
    E, D_OUT, _ = Wdiff.shape
    GK = orig_v.shape[2]
    TOP_K = 2
    T = B * S
    TB = 1024
    NB = T // TB

    x = hidden_states.reshape(T, D_IN)
    v_flat = jnp.transpose(orig_v, (1, 0, 2)).reshape(D_IN, E * GK)
    v16 = v_flat.astype(jnp.bfloat16)
    vt16 = v_flat.T.astype(jnp.bfloat16)
    w016 = W0.astype(jnp.bfloat16)
    wd16 = Wdiff.astype(jnp.bfloat16)
    b0r = b0.reshape(1, D_OUT)

    def run(assume_any):
        return pl.pallas_call(
            functools.partial(_moe_kernel, n_exp=E, gate_k=GK, top_k=TOP_K,
                              assume_any=assume_any),
            grid=(NB,),
            in_specs=[
                pl.BlockSpec((TB, D_IN), lambda i: (i, 0)),
                pl.BlockSpec((D_IN, E * GK), lambda i: (0, 0)),
                pl.BlockSpec((E * GK, D_IN), lambda i: (0, 0)),
                pl.BlockSpec((D_OUT, D_IN), lambda i: (0, 0)),
                pl.BlockSpec((1, D_OUT), lambda i: (0, 0)),
                pl.BlockSpec((E, D_OUT, D_IN), lambda i: (0, 0, 0)),
                pl.BlockSpec((E, D_OUT), lambda i: (0, 0)),
            ],
            out_specs=[
                pl.BlockSpec((TB, D_OUT), lambda i: (i, 0)),
                pl.BlockSpec((1, 1, 128), lambda i: (i, 0, 0)),
            ],
            out_shape=[
                jax.ShapeDtypeStruct((T, D_OUT), jnp.float32),
                jax.ShapeDtypeStruct((NB, 1, 128), jnp.float32),
            ],
            compiler_params=pltpu.CompilerParams(
                dimension_semantics=("parallel",)),
        )(x, v16, vt16, w016, b0r, wd16, bdiff)

    out_main, any_arr = run(True)
    any_flag = jnp.max(any_arr) > 0.0
    out = jax.lax.cond(any_flag,
                       lambda: out_main,
                       lambda: run(False)[0])
    return out.reshape(B, S, D_OUT)
